# Initial kernel scaffold; baseline (speedup 1.0000x reference)
import jax
import jax.numpy as jnp
from jax import lax
from jax.experimental import pallas as pl
from jax.experimental.pallas import tpu as pltpu
from jax.experimental.pallas import tpu_sc as plsc

NC = 2
NS = 16
L = 16
CHUNK = 128
D = 128
DH = 64


def _sc_layer_builder(e_pad, n_dst, n_res_pad, table_size):
    rows_total = e_pad // CHUNK
    rows_pt = rows_total // NS
    dst_stripe = n_dst // NS
    res_pt = n_res_pad // NS
    res_chunks = res_pt // CHUNK
    with_table = table_size is not None
    assert rows_total % NS == 0 and n_dst % NS == 0 and res_pt % CHUNK == 0
    assert dst_stripe % 125 == 0

    mesh = plsc.VectorSubcoreMesh(core_axis_name="c", subcore_axis_name="s",
                                  num_cores=NC, num_subcores=NS)
    out_type = [
        jax.ShapeDtypeStruct((NC, n_dst, DH), jnp.float32),
        jax.ShapeDtypeStruct((NC, n_dst, L), jnp.float32),
        jax.ShapeDtypeStruct((NC, n_res_pad, DH), jnp.float32),
    ]
    scratch = [
        pltpu.VMEM_SHARED((n_dst + 1, DH), jnp.float32),
        pltpu.VMEM_SHARED((n_dst + 1, L), jnp.float32),
        pltpu.VMEM((rows_pt * CHUNK,), jnp.int32),
        pltpu.VMEM((rows_pt, CHUNK), jnp.int32),
        pltpu.VMEM((CHUNK, DH), jnp.float32),
        pltpu.VMEM((CHUNK, L), jnp.float32),
        pltpu.VMEM((CHUNK, L), jnp.float32),
    ]
    if with_table:
        scratch.append(pltpu.VMEM((table_size,), jnp.int32))

    def body(src_hbm, dst2d_hbm, res_hbm, x2_hbm, *rest):
        if with_table:
            table_hbm = rest[0]
            (agg_out, deg_out, xdst_out, acc_sh, degacc_sh, srcbuf, dstbuf,
             rowbuf, onesbuf, zerosbuf, table_vm) = rest[1:]
        else:
            (agg_out, deg_out, xdst_out, acc_sh, degacc_sh, srcbuf, dstbuf,
             rowbuf, onesbuf, zerosbuf) = rest

        c = lax.axis_index("c")
        s = lax.axis_index("s")

        zv = jnp.zeros((L,), jnp.float32)
        ov = jnp.ones((L,), jnp.float32)

        def init_consts(i, carry):
            for j in range(DH // L):
                rowbuf[i, pl.ds(j * L, L)] = zv
            onesbuf[i, pl.ds(0, L)] = ov
            zerosbuf[i, pl.ds(0, L)] = zv
            return carry
        lax.fori_loop(0, CHUNK, init_consts, 0)

        def zero_stripe(i, carry):
            off = s * dst_stripe + i * 125
            pltpu.sync_copy(rowbuf.at[pl.ds(0, 125)],
                            acc_sh.at[pl.ds(off, 125)])
            pltpu.sync_copy(zerosbuf.at[pl.ds(0, 125)],
                            degacc_sh.at[pl.ds(off, 125)])
            return carry
        lax.fori_loop(0, dst_stripe // 125, zero_stripe, 0)

        if with_table:
            pltpu.sync_copy(table_hbm, table_vm)

        base = s * rows_pt
        pltpu.sync_copy(src_hbm.at[pl.ds(base * CHUNK, rows_pt * CHUNK)], srcbuf)
        pltpu.sync_copy(dst2d_hbm.at[pl.ds(base, rows_pt)], dstbuf)

        def compose(i, carry):
            v = srcbuf[pl.ds(i * L, L)]
            if with_table:
                v = plsc.load_gather(table_vm, [v])
            srcbuf[pl.ds(i * L, L)] = v * 2 + c
            return carry
        lax.fori_loop(0, rows_pt * CHUNK // L, compose, 0)

        plsc.subcore_barrier()

        def edge_step(j, carry):
            pltpu.sync_copy(x2_hbm.at[srcbuf.at[pl.ds(j * CHUNK, CHUNK)]],
                            rowbuf)
            pltpu.sync_copy(rowbuf, acc_sh.at[dstbuf.at[j]], add=True)
            pltpu.sync_copy(onesbuf, degacc_sh.at[dstbuf.at[j]], add=True)
            return carry
        lax.fori_loop(0, rows_pt, edge_step, 0)

        rbase = s * res_pt

        def res_step(j, carry):
            pltpu.sync_copy(res_hbm.at[pl.ds(rbase + j * CHUNK, CHUNK)],
                            srcbuf.at[pl.ds(0, CHUNK)])
            for k in range(CHUNK // L):
                v = srcbuf[pl.ds(k * L, L)]
                if with_table:
                    v = plsc.load_gather(table_vm, [v])
                srcbuf[pl.ds(k * L, L)] = v * 2 + c
            pltpu.sync_copy(x2_hbm.at[srcbuf.at[pl.ds(0, CHUNK)]], rowbuf)
            pltpu.sync_copy(rowbuf,
                            xdst_out.at[c, pl.ds(rbase + j * CHUNK, CHUNK)])
            return carry
        lax.fori_loop(0, res_chunks, res_step, 0)

        plsc.subcore_barrier()

        off = s * dst_stripe
        pltpu.sync_copy(acc_sh.at[pl.ds(off, dst_stripe)],
                        agg_out.at[c, pl.ds(off, dst_stripe)])
        pltpu.sync_copy(degacc_sh.at[pl.ds(off, dst_stripe)],
                        deg_out.at[c, pl.ds(off, dst_stripe)])

    return pl.kernel(body, out_type=out_type, mesh=mesh,
                     scratch_types=scratch)


def _tc_layer_builder(n_rows, block, final):
    grid = (n_rows // block,)

    def body(agg_ref, deg_ref, xdst_ref, w_ref, wr_ref, b_ref, o_ref):
        deg = jnp.maximum(deg_ref[0, :, 0:1], 1.0)
        inv = 1.0 / deg
        a_lo = agg_ref[0] * inv
        a_hi = agg_ref[1] * inv
        h = jnp.dot(a_lo, w_ref[0:DH, :], preferred_element_type=jnp.float32)
        h = h + jnp.dot(a_hi, w_ref[DH:D, :], preferred_element_type=jnp.float32)
        h = h + jnp.dot(xdst_ref[0], wr_ref[0:DH, :],
                        preferred_element_type=jnp.float32)
        h = h + jnp.dot(xdst_ref[1], wr_ref[DH:D, :],
                        preferred_element_type=jnp.float32)
        h = h + b_ref[0:1, :]
        if final:
            m = jnp.max(h, axis=1, keepdims=True)
            t = h - m
            lse = jnp.log(jnp.sum(jnp.exp(t), axis=1, keepdims=True))
            o_ref[...] = t - lse
        else:
            o_ref[...] = jnp.where(h > 0, h, jnp.exp(jnp.minimum(h, 0.0)) - 1.0)

    return pl.pallas_call(
        body,
        grid=grid,
        in_specs=[
            pl.BlockSpec((NC, block, DH), lambda i: (0, i, 0)),
            pl.BlockSpec((1, block, L), lambda i: (0, i, 0)),
            pl.BlockSpec((NC, block, DH), lambda i: (0, i, 0)),
            pl.BlockSpec((D, D), lambda i: (0, 0)),
            pl.BlockSpec((D, D), lambda i: (0, 0)),
            pl.BlockSpec((1, D), lambda i: (0, 0)),
        ],
        out_specs=pl.BlockSpec((block, D), lambda i: (i, 0)),
        out_shape=jax.ShapeDtypeStruct((n_rows, D), jnp.float32),
    )


def _pad_edges(src, dst, n_dst, e_pad):
    e = src.shape[0]
    pad = e_pad - e
    src_p = jnp.concatenate([src, jnp.zeros((pad,), jnp.int32)])
    dst_p = jnp.concatenate([dst, jnp.full((pad,), n_dst, jnp.int32)])
    return src_p, dst_p.reshape(e_pad // CHUNK, CHUNK)


def _pad_ids(ids, n_pad):
    return jnp.concatenate([ids, jnp.zeros((n_pad - ids.shape[0],), jnp.int32)])


def _round_up(n, m):
    return ((n + m - 1) // m) * m


def kernel(x, n_id, res_n_id1, edge_index1, res_n_id2, edge_index2,
           W1, W1_root, b1, W2, W2_root, b2):
    n_nodes = x.shape[0]
    n1 = n_id.shape[0]
    m1 = res_n_id1.shape[0]
    e1 = edge_index1.shape[1]
    m2 = res_n_id2.shape[0]
    e2 = edge_index2.shape[1]

    i32 = jnp.int32
    n_id = n_id.astype(i32)
    res_n_id1 = res_n_id1.astype(i32)
    res_n_id2 = res_n_id2.astype(i32)
    edge_index1 = edge_index1.astype(i32)
    edge_index2 = edge_index2.astype(i32)

    x2 = x.reshape(2 * n_nodes, DH)

    e1_pad = _round_up(e1, NS * CHUNK)
    m1_pad = _round_up(m1, NS * CHUNK)
    src1, dst1_2d = _pad_edges(edge_index1[0], edge_index1[1], m1, e1_pad)
    res1_p = _pad_ids(res_n_id1, m1_pad)
    sc1 = _sc_layer_builder(e1_pad, m1, m1_pad, n1)
    agg1, deg1, xdst1 = sc1(src1, dst1_2d, res1_p, x2, n_id)

    tc1 = _tc_layer_builder(m1, 2000, final=False)
    h1 = tc1(agg1, deg1, xdst1, W1, W1_root, b1.reshape(1, D))

    h1_2 = h1.reshape(2 * m1, DH)
    e2_pad = _round_up(e2, NS * CHUNK)
    m2_pad = _round_up(m2, NS * CHUNK)
    src2, dst2_2d = _pad_edges(edge_index2[0], edge_index2[1], m2, e2_pad)
    res2_p = _pad_ids(res_n_id2, m2_pad)
    sc2 = _sc_layer_builder(e2_pad, m2, m2_pad, None)
    agg2, deg2, xdst2 = sc2(src2, dst2_2d, res2_p, h1_2)

    tc2 = _tc_layer_builder(m2, 2000, final=True)
    return tc2(agg2, deg2, xdst2, W2, W2_root, b2.reshape(1, D))


# speedup vs baseline: 2.0075x; 2.0075x over previous
import jax
import jax.numpy as jnp
from jax import lax
from jax.experimental import pallas as pl
from jax.experimental.pallas import tpu as pltpu
from jax.experimental.pallas import tpu_sc as plsc

NC = 2
NS = 16
L = 16
CHUNK = 128
BLK = 8
D = 128


def _round_up(n, m):
    return ((n + m - 1) // m) * m


def _sc_layer_builder(e_pad, n_dst, n_res_pad, with_table):
    rows_total = e_pad // CHUNK
    rows_pt = rows_total // NS
    blocks_pt = rows_pt // BLK
    half = n_dst // 2
    h_pad = _round_up(half + 1, NS * 64)
    stripe = h_pad // NS
    res_rows_pw = (n_res_pad // CHUNK) // (NC * NS)
    assert n_dst % 2 == 0 and rows_pt % BLK == 0
    assert n_res_pad % (CHUNK * NC * NS) == 0 and stripe % 64 == 0

    mesh = plsc.VectorSubcoreMesh(core_axis_name="c", subcore_axis_name="s",
                                  num_cores=NC, num_subcores=NS)
    out_type = [
        jax.ShapeDtypeStruct((NC, h_pad, D), jnp.float32),
        jax.ShapeDtypeStruct((NC, h_pad, L), jnp.float32),
        jax.ShapeDtypeStruct((n_res_pad, D), jnp.float32),
    ]
    scratch = [
        pltpu.VMEM_SHARED((h_pad, D), jnp.float32),
        pltpu.VMEM_SHARED((h_pad, L), jnp.float32),
        pltpu.VMEM((BLK * CHUNK,), jnp.int32),
        pltpu.VMEM((BLK, CHUNK), jnp.int32),
        pltpu.VMEM((BLK, CHUNK), jnp.int32),
        pltpu.VMEM((CHUNK, D), jnp.float32),
        pltpu.VMEM((CHUNK, L), jnp.float32),
        pltpu.VMEM((64, L), jnp.float32),
    ]
    if with_table:
        scratch.append(pltpu.VMEM((CHUNK,), jnp.int32))

    def body(src_hbm, dst2d_hbm, res2d_hbm, x_hbm, *rest):
        if with_table:
            nid_hbm = rest[0]
            (agg_out, deg_out, xdst_out, acc_sh, degacc_sh, srcblk, dstblk,
             locblk, rowbuf, onesbuf, zerosbuf, nbuf) = rest[1:]
        else:
            (agg_out, deg_out, xdst_out, acc_sh, degacc_sh, srcblk, dstblk,
             locblk, rowbuf, onesbuf, zerosbuf) = rest

        c = lax.axis_index("c")
        s = lax.axis_index("s")
        lo = c * half

        zv = jnp.zeros((L,), jnp.float32)
        ov = jnp.ones((L,), jnp.float32)

        def init_consts(i, carry):
            for j in range(D // L):
                rowbuf[i, pl.ds(j * L, L)] = zv
            onesbuf[i, pl.ds(0, L)] = ov
            return carry
        lax.fori_loop(0, CHUNK, init_consts, 0)

        def init_zeros(i, carry):
            zerosbuf[i, pl.ds(0, L)] = zv
            return carry
        lax.fori_loop(0, 64, init_zeros, 0)

        def zero_stripe(i, carry):
            off = s * stripe + i * 64
            pltpu.sync_copy(rowbuf.at[pl.ds(0, 64)], acc_sh.at[pl.ds(off, 64)])
            pltpu.sync_copy(zerosbuf, degacc_sh.at[pl.ds(off, 64)])
            return carry
        lax.fori_loop(0, stripe // 64, zero_stripe, 0)

        plsc.subcore_barrier()

        base = s * rows_pt

        def edge_block(b, carry):
            blk0 = base + b * BLK
            pltpu.sync_copy(src_hbm.at[pl.ds(blk0 * CHUNK, BLK * CHUNK)],
                            srcblk)
            pltpu.sync_copy(dst2d_hbm.at[pl.ds(blk0, BLK)], dstblk)
            for jj in range(BLK):
                for k in range(CHUNK // L):
                    v = dstblk[jj, pl.ds(k * L, L)] - lo
                    sel = (v >= 0) & (v < half)
                    locblk[jj, pl.ds(k * L, L)] = jnp.where(sel, v, half)
            for jj in range(BLK):
                sidx = srcblk.at[pl.ds(jj * CHUNK, CHUNK)]
                if with_table:
                    pltpu.sync_copy(nid_hbm.at[sidx], nbuf)
                    pltpu.sync_copy(x_hbm.at[nbuf], rowbuf)
                else:
                    pltpu.sync_copy(x_hbm.at[sidx], rowbuf)
                pltpu.sync_copy(rowbuf, acc_sh.at[locblk.at[jj]], add=True)
                pltpu.sync_copy(onesbuf, degacc_sh.at[locblk.at[jj]], add=True)
            return carry
        lax.fori_loop(0, blocks_pt, edge_block, 0)

        w = s * NC + c
        rbase = w * res_rows_pw

        def res_step(j, carry):
            pltpu.sync_copy(res2d_hbm.at[pl.ds(rbase + j, 1)],
                            dstblk.at[pl.ds(0, 1)])
            ridx = dstblk.at[0]
            if with_table:
                pltpu.sync_copy(nid_hbm.at[ridx], nbuf)
                pltpu.sync_copy(x_hbm.at[nbuf], rowbuf)
            else:
                pltpu.sync_copy(x_hbm.at[ridx], rowbuf)
            pltpu.sync_copy(rowbuf, xdst_out.at[pl.ds((rbase + j) * CHUNK,
                                                      CHUNK)])
            return carry
        lax.fori_loop(0, res_rows_pw, res_step, 0)

        plsc.subcore_barrier()

        off = s * stripe
        pltpu.sync_copy(acc_sh.at[pl.ds(off, stripe)],
                        agg_out.at[c, pl.ds(off, stripe)])
        pltpu.sync_copy(degacc_sh.at[pl.ds(off, stripe)],
                        deg_out.at[c, pl.ds(off, stripe)])

    return pl.kernel(body, out_type=out_type, mesh=mesh,
                     scratch_types=scratch,
                     compiler_params=pltpu.CompilerParams(
                         needs_layout_passes=False,
                         use_tc_tiling_on_sc=False))


def _tc_layer_builder(n_dst, block, final):
    grid = (n_dst // block,)
    nb = (n_dst // 2) // block

    def body(agg_ref, deg_ref, xdst_ref, w_ref, wr_ref, b_ref, o_ref):
        deg = jnp.maximum(deg_ref[0, :, 0:1], 1.0)
        inv = 1.0 / deg
        h = jnp.dot(agg_ref[0] * inv, w_ref[...],
                    preferred_element_type=jnp.float32)
        h = h + jnp.dot(xdst_ref[...], wr_ref[...],
                        preferred_element_type=jnp.float32)
        h = h + b_ref[0:1, :]
        if final:
            m = jnp.max(h, axis=1, keepdims=True)
            t = h - m
            lse = jnp.log(jnp.sum(jnp.exp(t), axis=1, keepdims=True))
            o_ref[...] = t - lse
        else:
            o_ref[...] = jnp.where(h > 0, h, jnp.exp(jnp.minimum(h, 0.0)) - 1.0)

    return pl.pallas_call(
        body,
        grid=grid,
        in_specs=[
            pl.BlockSpec((1, block, D), lambda i: (i // nb, i % nb, 0)),
            pl.BlockSpec((1, block, L), lambda i: (i // nb, i % nb, 0)),
            pl.BlockSpec((block, D), lambda i: (i, 0)),
            pl.BlockSpec((D, D), lambda i: (0, 0)),
            pl.BlockSpec((D, D), lambda i: (0, 0)),
            pl.BlockSpec((1, D), lambda i: (0, 0)),
        ],
        out_specs=pl.BlockSpec((block, D), lambda i: (i, 0)),
        out_shape=jax.ShapeDtypeStruct((n_dst, D), jnp.float32),
    )


def _pad_edges(src, dst, n_dst, e_pad):
    e = src.shape[0]
    pad = e_pad - e
    src_p = jnp.concatenate([src, jnp.zeros((pad,), jnp.int32)])
    dst_p = jnp.concatenate([dst, jnp.full((pad,), n_dst, jnp.int32)])
    return src_p, dst_p.reshape(e_pad // CHUNK, CHUNK)


def _pad_ids_2d(ids, n_pad):
    p = jnp.concatenate([ids, jnp.zeros((n_pad - ids.shape[0],), jnp.int32)])
    return p.reshape(n_pad // CHUNK, CHUNK)


def kernel(x, n_id, res_n_id1, edge_index1, res_n_id2, edge_index2,
           W1, W1_root, b1, W2, W2_root, b2):
    m1 = res_n_id1.shape[0]
    e1 = edge_index1.shape[1]
    m2 = res_n_id2.shape[0]
    e2 = edge_index2.shape[1]

    i32 = jnp.int32
    n_id = n_id.astype(i32)
    res_n_id1 = res_n_id1.astype(i32)
    res_n_id2 = res_n_id2.astype(i32)
    edge_index1 = edge_index1.astype(i32)
    edge_index2 = edge_index2.astype(i32)

    e1_pad = _round_up(e1, NS * CHUNK * BLK)
    m1_pad = _round_up(m1, NC * NS * CHUNK)
    src1, dst1_2d = _pad_edges(edge_index1[0], edge_index1[1], m1, e1_pad)
    res1_2d = _pad_ids_2d(res_n_id1, m1_pad)
    sc1 = _sc_layer_builder(e1_pad, m1, m1_pad, True)
    agg1, deg1, xdst1 = sc1(src1, dst1_2d, res1_2d, x, n_id)

    tc1 = _tc_layer_builder(m1, 1000, final=False)
    h1 = tc1(agg1, deg1, xdst1, W1, W1_root, b1.reshape(1, D))

    e2_pad = _round_up(e2, NS * CHUNK * BLK)
    m2_pad = _round_up(m2, NC * NS * CHUNK)
    src2, dst2_2d = _pad_edges(edge_index2[0], edge_index2[1], m2, e2_pad)
    res2_2d = _pad_ids_2d(res_n_id2, m2_pad)
    sc2 = _sc_layer_builder(e2_pad, m2, m2_pad, False)
    agg2, deg2, xdst2 = sc2(src2, dst2_2d, res2_2d, h1)

    tc2 = _tc_layer_builder(m2, 1000, final=True)
    return tc2(agg2, deg2, xdst2, W2, W2_root, b2.reshape(1, D))


# speedup vs baseline: 2.3251x; 1.1582x over previous
import jax
import jax.numpy as jnp
from jax import lax
from jax.experimental import pallas as pl
from jax.experimental.pallas import tpu as pltpu
from jax.experimental.pallas import tpu_sc as plsc

NC = 2
NS = 16
L = 16
CH = 64
BLK = 8
D = 128


def _round_up(n, m):
    return ((n + m - 1) // m) * m


def _sc_layer_builder(e_pad, n_dst, n_res_pad, with_table):
    rows_total = e_pad // CH
    rows_pt = rows_total // NS
    blocks_pt = rows_pt // BLK
    half = n_dst // 2
    h_pad = _round_up(half + 1, NS * 64)
    stripe = h_pad // NS
    res_rows_pw = (n_res_pad // CH) // (NC * NS)
    assert n_dst % 2 == 0 and rows_pt % BLK == 0 and blocks_pt % 2 == 0
    assert n_res_pad % (CH * NC * NS) == 0 and stripe % 64 == 0

    mesh = plsc.VectorSubcoreMesh(core_axis_name="c", subcore_axis_name="s",
                                  num_cores=NC, num_subcores=NS)
    out_type = [
        jax.ShapeDtypeStruct((NC, h_pad, D), jnp.float32),
        jax.ShapeDtypeStruct((NC, h_pad, L), jnp.float32),
        jax.ShapeDtypeStruct((n_res_pad, D), jnp.float32),
    ]
    scratch = [
        pltpu.VMEM_SHARED((h_pad, D), jnp.float32),
        pltpu.VMEM_SHARED((h_pad, L), jnp.float32),
        pltpu.VMEM((2, BLK * CH), jnp.int32),
        pltpu.VMEM((2, BLK, CH), jnp.int32),
        pltpu.VMEM((2, BLK, CH), jnp.int32),
        pltpu.VMEM((BLK, CH), jnp.int32),
        pltpu.VMEM((CH, D), jnp.float32),
        pltpu.VMEM((CH, D), jnp.float32),
        pltpu.VMEM((CH, L), jnp.float32),
        pltpu.VMEM((64, L), jnp.float32),
        pltpu.VMEM((res_rows_pw, CH), jnp.int32),
        pltpu.VMEM((res_rows_pw, CH), jnp.int32),
        pltpu.SemaphoreType.DMA,
        pltpu.SemaphoreType.DMA,
        pltpu.SemaphoreType.DMA,
        pltpu.SemaphoreType.DMA,
        pltpu.SemaphoreType.DMA,
        pltpu.SemaphoreType.DMA,
        pltpu.SemaphoreType.DMA,
        pltpu.SemaphoreType.DMA,
    ]

    def body(src_hbm, dst2d_hbm, res3d_hbm, x_hbm, *rest):
        if with_table:
            nid_hbm = rest[0]
            rest = rest[1:]
        (agg_out, deg_out, xdst_out, acc_sh, degacc_sh, srcblk, dstblk,
         locblk, nbuf, rowA, rowB, onesbuf, zerosbuf, resblk, nbufres,
         stsem, nsem, gsemA, gsemB, ssemA, ssemB, dsem, wsem) = rest

        c = lax.axis_index("c")
        s = lax.axis_index("s")
        lo = c * half
        rows = (rowA, rowB)
        gsems = (gsemA, gsemB)
        ssems = (ssemA, ssemB)

        zv = jnp.zeros((L,), jnp.float32)
        ov = jnp.ones((L,), jnp.float32)

        def init_consts(i, carry):
            for j in range(D // L):
                rowA[i, pl.ds(j * L, L)] = zv
            onesbuf[i, pl.ds(0, L)] = ov
            zerosbuf[i, pl.ds(0, L)] = zv
            return carry
        lax.fori_loop(0, CH, init_consts, 0)

        def zero_stripe(i, carry):
            off = s * stripe + i * 64
            pltpu.sync_copy(rowA.at[pl.ds(0, 64)], acc_sh.at[pl.ds(off, 64)])
            pltpu.sync_copy(zerosbuf, degacc_sh.at[pl.ds(off, 64)])
            return carry
        lax.fori_loop(0, stripe // 64, zero_stripe, 0)

        plsc.subcore_barrier()

        base = s * rows_pt

        def fire_stage(bb, buf_i):
            blk0 = base + bb * BLK
            pltpu.async_copy(src_hbm.at[pl.ds(blk0 * CH, BLK * CH)],
                             srcblk.at[buf_i], stsem)
            pltpu.async_copy(dst2d_hbm.at[pl.ds(blk0, BLK)],
                             dstblk.at[buf_i], stsem)

        def wait_stage():
            pltpu.make_async_copy(src_hbm.at[pl.ds(0, BLK * CH)],
                                  srcblk.at[0], stsem).wait()
            pltpu.make_async_copy(dst2d_hbm.at[pl.ds(0, BLK)],
                                  dstblk.at[0], stsem).wait()

        def wait_sc(p):
            pltpu.make_async_copy(
                rowA, acc_sh.at[locblk.at[0, 0]], ssems[p]).wait()

        def wait_deg():
            pltpu.make_async_copy(
                onesbuf, degacc_sh.at[locblk.at[0, 0]], dsem).wait()

        def wait_xg(p):
            pltpu.make_async_copy(
                x_hbm.at[nbuf.at[0]], rowA, gsems[p]).wait()

        def wait_nid(dst):
            pltpu.make_async_copy(
                nid_hbm.at[srcblk.at[0, pl.ds(0, CH)]], dst, nsem).wait()

        fire_stage(jnp.int32(0), 0)

        def block_pair(t, carry):
            for half_i in range(2):
                bb = t * 2 + half_i
                sb = srcblk.at[half_i]
                db = dstblk.at[half_i]
                lb = locblk.at[half_i]
                wait_stage()
                fire_stage(jnp.minimum(bb + 1, blocks_pt - 1), 1 - half_i)

                @pl.when(bb > 0)
                def _():
                    for _jj in range(BLK):
                        wait_deg()

                for jj in range(BLK):
                    for k in range(CH // L):
                        v = db[jj, pl.ds(k * L, L)] - lo
                        sel = (v >= 0) & (v < half)
                        lb[jj, pl.ds(k * L, L)] = jnp.where(sel, v, half)

                if with_table:
                    for jj in range(BLK):
                        pltpu.async_copy(
                            nid_hbm.at[sb.at[pl.ds(jj * CH, CH)]],
                            nbuf.at[jj], nsem)
                    for jj in range(BLK):
                        wait_nid(nbuf.at[jj])

                for jj in range(BLK):
                    if jj >= 2:
                        wait_sc(jj % 2)
                    else:
                        @pl.when(bb > 0)
                        def _():
                            wait_sc(jj % 2)
                    if with_table:
                        pltpu.async_copy(x_hbm.at[nbuf.at[jj]],
                                         rows[jj % 2], gsems[jj % 2])
                    else:
                        pltpu.async_copy(x_hbm.at[sb.at[pl.ds(jj * CH, CH)]],
                                         rows[jj % 2], gsems[jj % 2])
                    if jj >= 1:
                        wait_xg((jj - 1) % 2)
                        pltpu.async_copy(rows[(jj - 1) % 2],
                                         acc_sh.at[lb.at[jj - 1]],
                                         ssems[(jj - 1) % 2], add=True)
                        pltpu.async_copy(onesbuf,
                                         degacc_sh.at[lb.at[jj - 1]], dsem,
                                         add=True)
                wait_xg((BLK - 1) % 2)
                pltpu.async_copy(rows[(BLK - 1) % 2],
                                 acc_sh.at[lb.at[BLK - 1]],
                                 ssems[(BLK - 1) % 2], add=True)
                pltpu.async_copy(onesbuf, degacc_sh.at[lb.at[BLK - 1]], dsem,
                                 add=True)
            return carry
        lax.fori_loop(0, blocks_pt // 2, block_pair, 0)

        wait_sc(0)
        wait_sc(1)
        for _jj in range(BLK):
            wait_deg()
        wait_stage()

        w = s * NC + c
        pltpu.sync_copy(res3d_hbm.at[w], resblk)
        if with_table:
            for j in range(res_rows_pw):
                pltpu.async_copy(nid_hbm.at[resblk.at[j]], nbufres.at[j],
                                 nsem)
            for j in range(res_rows_pw):
                wait_nid(nbufres.at[j])

        def res_idx(j):
            return nbufres.at[j] if with_table else resblk.at[j]

        def wait_wout():
            pltpu.make_async_copy(
                rowA, xdst_out.at[pl.ds(0, CH)], wsem).wait()

        for j in range(res_rows_pw):
            if j >= 2:
                wait_wout()
            pltpu.async_copy(x_hbm.at[res_idx(j)], rows[j % 2],
                             gsems[j % 2])
            if j >= 1:
                wait_xg((j - 1) % 2)
                pltpu.async_copy(
                    rows[(j - 1) % 2],
                    xdst_out.at[pl.ds((w * res_rows_pw + j - 1) * CH, CH)],
                    wsem)
        wait_xg((res_rows_pw - 1) % 2)
        pltpu.async_copy(
            rows[(res_rows_pw - 1) % 2],
            xdst_out.at[pl.ds((w * res_rows_pw + res_rows_pw - 1) * CH, CH)],
            wsem)
        wait_wout()
        if res_rows_pw > 1:
            wait_wout()

        plsc.subcore_barrier()

        off = s * stripe
        pltpu.sync_copy(acc_sh.at[pl.ds(off, stripe)],
                        agg_out.at[c, pl.ds(off, stripe)])
        pltpu.sync_copy(degacc_sh.at[pl.ds(off, stripe)],
                        deg_out.at[c, pl.ds(off, stripe)])

    return pl.kernel(body, out_type=out_type, mesh=mesh,
                     scratch_types=scratch,
                     compiler_params=pltpu.CompilerParams(
                         needs_layout_passes=False,
                         use_tc_tiling_on_sc=False))


def _tc_layer_builder(n_dst, block, final):
    grid = (n_dst // block,)
    nb = (n_dst // 2) // block

    def body(agg_ref, deg_ref, xdst_ref, w_ref, wr_ref, b_ref, o_ref):
        deg = jnp.maximum(deg_ref[0, :, 0:1], 1.0)
        inv = 1.0 / deg
        h = jnp.dot(agg_ref[0] * inv, w_ref[...],
                    preferred_element_type=jnp.float32)
        h = h + jnp.dot(xdst_ref[...], wr_ref[...],
                        preferred_element_type=jnp.float32)
        h = h + b_ref[0:1, :]
        if final:
            m = jnp.max(h, axis=1, keepdims=True)
            t = h - m
            lse = jnp.log(jnp.sum(jnp.exp(t), axis=1, keepdims=True))
            o_ref[...] = t - lse
        else:
            o_ref[...] = jnp.where(h > 0, h, jnp.exp(jnp.minimum(h, 0.0)) - 1.0)

    return pl.pallas_call(
        body,
        grid=grid,
        in_specs=[
            pl.BlockSpec((1, block, D), lambda i: (i // nb, i % nb, 0)),
            pl.BlockSpec((1, block, L), lambda i: (i // nb, i % nb, 0)),
            pl.BlockSpec((block, D), lambda i: (i, 0)),
            pl.BlockSpec((D, D), lambda i: (0, 0)),
            pl.BlockSpec((D, D), lambda i: (0, 0)),
            pl.BlockSpec((1, D), lambda i: (0, 0)),
        ],
        out_specs=pl.BlockSpec((block, D), lambda i: (i, 0)),
        out_shape=jax.ShapeDtypeStruct((n_dst, D), jnp.float32),
    )


def _pad_edges(src, dst, n_dst, e_pad):
    e = src.shape[0]
    pad = e_pad - e
    src_p = jnp.concatenate([src, jnp.zeros((pad,), jnp.int32)])
    dst_p = jnp.concatenate([dst, jnp.full((pad,), n_dst, jnp.int32)])
    return src_p, dst_p.reshape(e_pad // CH, CH)


def _pad_ids_3d(ids, n_pad):
    p = jnp.concatenate([ids, jnp.zeros((n_pad - ids.shape[0],), jnp.int32)])
    return p.reshape(NC * NS, (n_pad // CH) // (NC * NS), CH)


def kernel(x, n_id, res_n_id1, edge_index1, res_n_id2, edge_index2,
           W1, W1_root, b1, W2, W2_root, b2):
    m1 = res_n_id1.shape[0]
    e1 = edge_index1.shape[1]
    m2 = res_n_id2.shape[0]
    e2 = edge_index2.shape[1]

    i32 = jnp.int32
    n_id = n_id.astype(i32)
    res_n_id1 = res_n_id1.astype(i32)
    res_n_id2 = res_n_id2.astype(i32)
    edge_index1 = edge_index1.astype(i32)
    edge_index2 = edge_index2.astype(i32)

    e1_pad = _round_up(e1, NS * CH * BLK * 2)
    m1_pad = _round_up(m1, NC * NS * CH)
    src1, dst1_2d = _pad_edges(edge_index1[0], edge_index1[1], m1, e1_pad)
    res1_3d = _pad_ids_3d(res_n_id1, m1_pad)
    sc1 = _sc_layer_builder(e1_pad, m1, m1_pad, True)
    agg1, deg1, xdst1 = sc1(src1, dst1_2d, res1_3d, x, n_id)

    tc1 = _tc_layer_builder(m1, 1000, final=False)
    h1 = tc1(agg1, deg1, xdst1, W1, W1_root, b1.reshape(1, D))

    e2_pad = _round_up(e2, NS * CH * BLK * 2)
    m2_pad = _round_up(m2, NC * NS * CH)
    src2, dst2_2d = _pad_edges(edge_index2[0], edge_index2[1], m2, e2_pad)
    res2_3d = _pad_ids_3d(res_n_id2, m2_pad)
    sc2 = _sc_layer_builder(e2_pad, m2, m2_pad, False)
    agg2, deg2, xdst2 = sc2(src2, dst2_2d, res2_3d, h1)

    tc2 = _tc_layer_builder(m2, 1000, final=True)
    return tc2(agg2, deg2, xdst2, W2, W2_root, b2.reshape(1, D))
